# Initial kernel scaffold; baseline (speedup 1.0000x reference)
import jax
import jax.numpy as jnp
from jax.experimental import pallas as pl


def kernel(weight_tensor, index_tensor):
    raise NotImplementedError("write your pallas kernel here")


# speedup vs baseline: 1.4363x; 1.4363x over previous
import functools

import jax
import jax.numpy as jnp
from jax import lax
from jax.experimental import pallas as pl
from jax.experimental.pallas import tpu as pltpu
from jax.experimental.pallas import tpu_sc as plsc

_NC = 2
_NS = 16
_NW = _NC * _NS
_CHUNK = 128


def _gather_body(n_chunks, d, table_hbm, idx_hbm, out_hbm, idx_v, rows_v, sem):
    wid = lax.axis_index("s") * _NC + lax.axis_index("c")
    base = wid * (n_chunks * _CHUNK)
    pltpu.sync_copy(idx_hbm.at[wid], idx_v)

    def body(j, carry):
        pltpu.async_copy(table_hbm.at[idx_v.at[j]], rows_v, sem).wait()
        pltpu.sync_copy(rows_v, out_hbm.at[pl.ds(base + j * _CHUNK, _CHUNK)])
        return carry

    lax.fori_loop(0, n_chunks, body, 0)


@functools.partial(jax.jit, static_argnames=("n_chunks", "d"))
def _gather(table, idx, n_chunks, d):
    mesh = plsc.VectorSubcoreMesh(core_axis_name="c", subcore_axis_name="s")
    kfn = pl.kernel(
        functools.partial(_gather_body, n_chunks, d),
        out_type=jax.ShapeDtypeStruct((_NW * n_chunks * _CHUNK, d), table.dtype),
        mesh=mesh,
        scratch_types=[
            pltpu.VMEM((n_chunks, _CHUNK), jnp.int32),
            pltpu.VMEM((_CHUNK, d), table.dtype),
            pltpu.SemaphoreType.DMA,
        ],
        compiler_params=pltpu.CompilerParams(use_tc_tiling_on_sc=False),
    )
    return kfn(table, idx)


def kernel(weight_tensor, index_tensor):
    b0, b1 = index_tensor.shape
    d = weight_tensor.shape[1]
    n = b0 * b1
    assert n % (_NW * _CHUNK) == 0
    n_chunks = n // (_NW * _CHUNK)
    idx = index_tensor.astype(jnp.int32).reshape(_NW, n_chunks, _CHUNK)
    out = _gather(weight_tensor, idx, n_chunks, d)
    return out.reshape(b0, b1, d)


# speedup vs baseline: 1.5520x; 1.0805x over previous
import functools

import jax
import jax.numpy as jnp
from jax import lax
from jax.experimental import pallas as pl
from jax.experimental.pallas import tpu as pltpu
from jax.experimental.pallas import tpu_sc as plsc

_NC = 2
_NS = 16
_NW = _NC * _NS
_CHUNK = 512
_G = 1


def _gather_body(n_groups, d, table_hbm, idx_hbm, out_hbm,
                 idx_v, rows0, rows1, gsem0, gsem1, ssem0, ssem1):
    wid = lax.axis_index("s") * _NC + lax.axis_index("c")
    group_rows = _G * _CHUNK
    base = wid * (n_groups * group_rows)
    rows = (rows0, rows1)
    gsem = (gsem0, gsem1)
    ssem = (ssem0, ssem1)
    pltpu.sync_copy(idx_hbm.at[wid], idx_v)

    def pair(t, carry):
        for b in range(2):
            g = 2 * t + b
            @pl.when(t >= 1)
            def _():
                pltpu.make_async_copy(
                    rows[b], out_hbm.at[pl.ds(0, group_rows)], ssem[b]).wait()

            handles = [
                pltpu.async_copy(
                    table_hbm.at[idx_v.at[g * _G + u]],
                    rows[b].at[pl.ds(u * _CHUNK, _CHUNK)], gsem[b])
                for u in range(_G)
            ]
            for h in handles:
                h.wait()
            pltpu.async_copy(
                rows[b], out_hbm.at[pl.ds(base + g * group_rows, group_rows)],
                ssem[b])
        return carry

    lax.fori_loop(0, n_groups // 2, pair, 0)
    for b in range(2):
        pltpu.make_async_copy(
            rows[b], out_hbm.at[pl.ds(0, group_rows)], ssem[b]).wait()


@functools.partial(jax.jit, static_argnames=("n_groups", "d"))
def _gather(table, idx, n_groups, d):
    mesh = plsc.VectorSubcoreMesh(core_axis_name="c", subcore_axis_name="s")
    n_chunks = n_groups * _G
    kfn = pl.kernel(
        functools.partial(_gather_body, n_groups, d),
        out_type=jax.ShapeDtypeStruct((_NW * n_chunks * _CHUNK, d), table.dtype),
        mesh=mesh,
        scratch_types=[
            pltpu.VMEM((n_chunks, _CHUNK), jnp.int32),
            pltpu.VMEM((_G * _CHUNK, d), table.dtype),
            pltpu.VMEM((_G * _CHUNK, d), table.dtype),
            pltpu.SemaphoreType.DMA,
            pltpu.SemaphoreType.DMA,
            pltpu.SemaphoreType.DMA,
            pltpu.SemaphoreType.DMA,
        ],
        compiler_params=pltpu.CompilerParams(use_tc_tiling_on_sc=False),
    )
    return kfn(table, idx)


def kernel(weight_tensor, index_tensor):
    b0, b1 = index_tensor.shape
    d = weight_tensor.shape[1]
    n = b0 * b1
    per_group = _NW * _G * _CHUNK
    assert n % (2 * per_group) == 0
    n_groups = n // per_group
    idx = index_tensor.astype(jnp.int32).reshape(_NW, n_groups * _G, _CHUNK)
    out = _gather(weight_tensor, idx, n_groups, d)
    return out.reshape(b0, b1, d)


# speedup vs baseline: 1.5765x; 1.0158x over previous
import functools

import jax
import jax.numpy as jnp
from jax import lax
from jax.experimental import pallas as pl
from jax.experimental.pallas import tpu as pltpu
from jax.experimental.pallas import tpu_sc as plsc

_NC = 2
_NS = 16
_NW = _NC * _NS
_CHUNK = 512
_G = 1


def _gather_body(n_groups, d, table_hbm, idx_hbm, out_hbm,
                 idx_v, rows0, rows1, gsem0, gsem1, ssem0, ssem1):
    wid = lax.axis_index("s") * _NC + lax.axis_index("c")
    group_rows = _G * _CHUNK
    base = wid * (n_groups * group_rows)
    rows = (rows0, rows1)
    gsem = (gsem0, gsem1)
    ssem = (ssem0, ssem1)
    pltpu.sync_copy(idx_hbm.at[wid], idx_v)

    def pair(t, carry):
        for b in range(2):
            g = 2 * t + b
            @pl.when(t >= 1)
            def _():
                pltpu.make_async_copy(
                    rows[b], out_hbm.at[pl.ds(0, group_rows)], ssem[b]).wait()

            handles = [
                pltpu.async_copy(
                    table_hbm.at[idx_v.at[g * _G + u]],
                    rows[b].at[pl.ds(u * _CHUNK, _CHUNK)], gsem[b])
                for u in range(_G)
            ]
            for h in handles:
                h.wait()
            pltpu.async_copy(
                rows[b], out_hbm.at[pl.ds(base + g * group_rows, group_rows)],
                ssem[b])
        return carry

    lax.fori_loop(0, n_groups // 2, pair, 0)
    for b in range(2):
        pltpu.make_async_copy(
            rows[b], out_hbm.at[pl.ds(0, group_rows)], ssem[b]).wait()


@functools.partial(jax.jit, static_argnames=("n_groups", "d"))
def _gather(table, idx, n_groups, d):
    mesh = plsc.VectorSubcoreMesh(core_axis_name="c", subcore_axis_name="s")
    n_chunks = n_groups * _G
    kfn = pl.kernel(
        functools.partial(_gather_body, n_groups, d),
        out_type=jax.ShapeDtypeStruct((_NW * n_chunks * _CHUNK, d), table.dtype),
        mesh=mesh,
        scratch_types=[
            pltpu.VMEM((n_chunks, _CHUNK), jnp.int32),
            pltpu.VMEM((_G * _CHUNK, d), table.dtype),
            pltpu.VMEM((_G * _CHUNK, d), table.dtype),
            pltpu.SemaphoreType.DMA,
            pltpu.SemaphoreType.DMA,
            pltpu.SemaphoreType.DMA,
            pltpu.SemaphoreType.DMA,
        ],
        compiler_params=pltpu.CompilerParams(use_tc_tiling_on_sc=False),
    )
    return kfn(table, idx)


def kernel(weight_tensor, index_tensor):
    b0, b1 = index_tensor.shape
    v, d = weight_tensor.shape
    n = b0 * b1
    per_group = _NW * _G * _CHUNK
    assert n % (2 * per_group) == 0
    n_groups = n // per_group
    pad = 128 // d
    wp = jnp.pad(weight_tensor, ((0, 0), (0, 128 - d))).reshape(v * pad, d)
    idx = index_tensor.astype(jnp.int32).reshape(_NW, n_groups * _G, _CHUNK)
    out = _gather(wp, idx * pad, n_groups, d)
    return out.reshape(b0, b1, d)
